# Initial kernel scaffold; baseline (speedup 1.0000x reference)
import jax
import jax.numpy as jnp
from jax.experimental import pallas as pl


def kernel(seeds, es, neighbors):
    raise NotImplementedError("write your pallas kernel here")


# speedup vs baseline: 7.2724x; 7.2724x over previous
import functools

import jax
import jax.numpy as jnp
from jax.experimental import pallas as pl
from jax.experimental.pallas import tpu as pltpu

N_CLASS = 8
SEED_COUNT = 16
STEP = 3
MIN_MATCH = 3
N = 10000
D = 256


def _sims_body(cat_ref, es_ref, out_ref):
    x = es_ref[...]
    ss = jnp.sum(x * x, axis=1, keepdims=True)
    nrm = jnp.sqrt(ss)
    xn = x / (nrm + 1e-8)
    c = cat_ref[...]
    s = jax.lax.dot_general(c, xn, (((1,), (1,)), ((), ())),
                            preferred_element_type=jnp.float32)
    out_ref[...] = s * 0.5 + 0.5


def _sims(cat_n, es):
    return pl.pallas_call(
        _sims_body,
        out_shape=jax.ShapeDtypeStruct((N_CLASS, N), jnp.float32),
    )(cat_n, es)


def _counts_body(mask_ref, nbr_ref, out_ref):
    m = mask_ref[...]
    nb = nbr_ref[...]
    out_ref[...] = jax.lax.dot_general(
        m, nb, (((1,), (1,)), ((), ())), preferred_element_type=jnp.float32)


def _counts(mask, neighbors):
    ti = 512
    grid = (N + ti - 1) // ti
    return pl.pallas_call(
        _counts_body,
        grid=(grid,),
        in_specs=[
            pl.BlockSpec((N_CLASS, N), lambda i: (0, 0)),
            pl.BlockSpec((ti, N), lambda i: (i, 0)),
        ],
        out_specs=pl.BlockSpec((N_CLASS, ti), lambda i: (0, i)),
        out_shape=jax.ShapeDtypeStruct((N_CLASS, N), jnp.float32),
    )(mask, neighbors)


def _topk_body(mm, sims_ref, counts_ref, ent_ref, sel_ref, probs_ref, m_ref):
    sims = sims_ref[...]
    counts = counts_ref[...]
    ent = ent_ref[...]
    valid = jnp.logical_and(counts > mm, ent == 0.0)
    pools = jnp.max(jnp.where(valid, 1.0, 0.0), axis=0, keepdims=True)
    m_ref[:, :N] = sims * pools
    scores = jnp.where(valid, sims, -1.0)
    iot = jax.lax.broadcasted_iota(jnp.int32, (N_CLASS, N), 1)
    for r in range(SEED_COUNT):
        mx = jnp.max(scores, axis=1, keepdims=True)
        hit = scores == mx
        idx = jnp.min(jnp.where(hit, iot, jnp.int32(2**30)),
                      axis=1, keepdims=True)
        sel_ref[:, r:r + 1] = idx
        scores = jnp.where(iot == idx, -2.0, scores)
    lane = jax.lax.broadcasted_iota(jnp.int32, (1, 128), 1)
    for c in range(N_CLASS):
        for r in range(SEED_COUNT):
            i_cr = sel_ref[c, r]
            base = pl.multiple_of((i_cr // 128) * 128, 128)
            win = m_ref[:, pl.ds(base, 128)]
            col = jnp.sum(jnp.where(lane == i_cr - base, win, 0.0),
                          axis=1, keepdims=True)
            probs_ref[:, c * SEED_COUNT + r:c * SEED_COUNT + r + 1] = col


def _topk(sims, counts, ent, mm):
    return pl.pallas_call(
        functools.partial(_topk_body, float(mm)),
        out_shape=[
            jax.ShapeDtypeStruct((N_CLASS, SEED_COUNT), jnp.int32),
            jax.ShapeDtypeStruct((N_CLASS, N_CLASS * SEED_COUNT), jnp.float32),
        ],
        scratch_shapes=[pltpu.VMEM((N_CLASS, 10112), jnp.float32)],
    )(sims, counts, ent)


def kernel(seeds, es, neighbors):
    es = es.astype(jnp.float32)
    neighbors = neighbors.astype(jnp.float32)
    cat = jnp.mean(es[seeds].reshape(N_CLASS, SEED_COUNT, D), axis=1)
    cat_n = cat / (jnp.linalg.norm(cat, axis=-1, keepdims=True) + 1e-8)
    sims = _sims(cat_n, es)

    cvec = jnp.repeat(jnp.arange(N_CLASS, dtype=jnp.int32), SEED_COUNT)
    mask = jnp.zeros((N_CLASS, N), jnp.float32).at[cvec, seeds].set(1.0)
    ent = jnp.zeros((1, N), jnp.float32).at[0, seeds].set(1.0)

    probs_steps, sel_steps = [], []
    for rnn_i in range(STEP):
        mm = max(2, MIN_MATCH - rnn_i)
        counts = _counts(mask, neighbors)
        sel, probs8 = _topk(sims, counts, ent, mm)
        sel_flat = sel.reshape(-1)
        probs_steps.append(probs8.T)
        sel_steps.append(sel_flat)
        mask = mask.at[cvec, sel_flat].set(1.0)
        ent = ent.at[0, sel_flat].set(1.0)

    steps = jnp.full((STEP, N_CLASS), SEED_COUNT, dtype=jnp.int32)
    return (jnp.stack(probs_steps), jnp.stack(sel_steps), steps)
